# Initial kernel scaffold; baseline (speedup 1.0000x reference)
import jax
import jax.numpy as jnp
from jax.experimental import pallas as pl


def kernel(x, edge_index, edge_weight, energy, W, b):
    raise NotImplementedError("write your pallas kernel here")


# speedup vs baseline: 12.0727x; 12.0727x over previous
import dataclasses
import functools

import jax
import jax.numpy as jnp
from jax import lax
from jax.experimental import pallas as pl
from jax.experimental.pallas import tpu as pltpu
from jax.experimental.pallas import tpu_sc as plsc

N = 10000
E = 320000
D = 128

NC = 2
NS = 16
LANES = 16
NW = NC * NS
EPW = E // NW
K = 80
NCHUNK = EPW // K
NPAD = 10240
ROWS_PER_SUB = NPAD // NS
EPAD = NPAD

_mesh = plsc.VectorSubcoreMesh(
    core_axis_name="c", subcore_axis_name="s", num_cores=NC, num_subcores=NS
)


def _sc_body(h_hbm, e_hbm, src_hbm, dst_hbm, ew_hbm, zeros_hbm, out_hbm,
             e_loc, src_v, dst_v, ew_v, wn_v, rows_v, acc, sem):
    cid = lax.axis_index("c")
    sid = lax.axis_index("s")
    wid = cid * NS + sid

    pltpu.sync_copy(zeros_hbm, acc.at[pl.ds(sid * ROWS_PER_SUB, ROWS_PER_SUB)])

    pltpu.sync_copy(e_hbm, e_loc)

    @pl.loop(0, EPAD // LANES)
    def _clip(i):
        sl = pl.ds(i * LANES, LANES)
        e_loc[sl] = jnp.minimum(jnp.maximum(e_loc[sl], 0.0), 10.0)

    plsc.subcore_barrier()

    ebase = wid * EPW

    @pl.loop(0, NCHUNK)
    def _chunk(c):
        base = ebase + c * K
        pltpu.sync_copy(src_hbm.at[pl.ds(base, K)], src_v)
        pltpu.sync_copy(dst_hbm.at[pl.ds(base, K)], dst_v)
        pltpu.sync_copy(ew_hbm.at[pl.ds(base, K)], ew_v)
        pltpu.async_copy(h_hbm.at[src_v], rows_v, sem).wait()

        for v in range(K // LANES):
            sl = pl.ds(v * LANES, LANES)
            es = plsc.load_gather(e_loc, [src_v[sl]])
            ed = plsc.load_gather(e_loc, [dst_v[sl]])
            wn_v[sl] = ew_v[sl] / (1.0 + jnp.exp(es + ed))

        @pl.loop(0, K)
        def _scale(k):
            wk = plsc.load_gather(wn_v, [jnp.full((LANES,), k, jnp.int32)])
            for j in range(D // LANES):
                sl2 = pl.ds(j * LANES, LANES)
                rows_v[k, sl2] = rows_v[k, sl2] * wk

        pltpu.sync_copy(rows_v, acc.at[dst_v], add=True)

    plsc.subcore_barrier()

    sl = pl.ds(sid * ROWS_PER_SUB, ROWS_PER_SUB)
    pltpu.sync_copy(acc.at[sl], out_hbm.at[cid].at[sl])


_SC_COMPILER_PARAMS = pltpu.CompilerParams()
if "needs_layout_passes" in pltpu.CompilerParams.__dataclass_fields__:
    _SC_COMPILER_PARAMS = dataclasses.replace(
        _SC_COMPILER_PARAMS, needs_layout_passes=False
    )

_sc_call = pl.kernel(
    _sc_body,
    out_type=jax.ShapeDtypeStruct((NC, NPAD, D), jnp.float32),
    mesh=_mesh,
    scratch_types=[
        pltpu.VMEM((EPAD,), jnp.float32),
        pltpu.VMEM((K,), jnp.int32),
        pltpu.VMEM((K,), jnp.int32),
        pltpu.VMEM((K,), jnp.float32),
        pltpu.VMEM((K,), jnp.float32),
        pltpu.VMEM((K, D), jnp.float32),
        pltpu.VMEM_SHARED((NPAD, D), jnp.float32),
        pltpu.SemaphoreType.DMA,
    ],
    compiler_params=_SC_COMPILER_PARAMS,
)


_MM_BLK = 1000


def _matmul_body(x_ref, wt_ref, b_ref, h_ref):
    h_ref[...] = (
        jnp.dot(x_ref[...], wt_ref[...], preferred_element_type=jnp.float32)
        + b_ref[...]
    )


_matmul = pl.pallas_call(
    _matmul_body,
    grid=(N // _MM_BLK,),
    in_specs=[
        pl.BlockSpec((_MM_BLK, D), lambda i: (i, 0)),
        pl.BlockSpec((D, D), lambda i: (0, 0)),
        pl.BlockSpec((1, D), lambda i: (0, 0)),
    ],
    out_specs=pl.BlockSpec((_MM_BLK, D), lambda i: (i, 0)),
    out_shape=jax.ShapeDtypeStruct((N, D), jnp.float32),
)


def _combine_body(p_ref, o_ref):
    o_ref[...] = p_ref[0] + p_ref[1]


_CB_BLK = 1024

_combine = pl.pallas_call(
    _combine_body,
    grid=(NPAD // _CB_BLK,),
    in_specs=[pl.BlockSpec((NC, _CB_BLK, D), lambda i: (0, i, 0))],
    out_specs=pl.BlockSpec((_CB_BLK, D), lambda i: (i, 0)),
    out_shape=jax.ShapeDtypeStruct((NPAD, D), jnp.float32),
)


@jax.jit
def kernel(x, edge_index, edge_weight, energy, W, b):
    src = edge_index[0].astype(jnp.int32)
    dst = edge_index[1].astype(jnp.int32)
    ew = edge_weight.astype(jnp.float32)
    e_pad = jnp.pad(energy.reshape(-1), (0, EPAD - N))
    zeros = jnp.zeros((ROWS_PER_SUB, D), jnp.float32)
    h = _matmul(x, W.T, b.reshape(1, D))
    partials = _sc_call(h, e_pad, src, dst, ew, zeros)
    return _combine(partials)[:N]


# speedup vs baseline: 23.3259x; 1.9321x over previous
import dataclasses

import jax
import jax.numpy as jnp
from jax import lax
from jax.experimental import pallas as pl
from jax.experimental.pallas import tpu as pltpu
from jax.experimental.pallas import tpu_sc as plsc

N = 10000
E = 320000
D = 128

NC = 2
NS = 16
LANES = 16
NW = NC * NS
EPW = E // NW
K = 80
NCHUNK = EPW // K
ECHUNK = E // K
NPAD = 10240
ROWS_PER_SUB = NPAD // NS
EPAD = NPAD

_mesh = plsc.VectorSubcoreMesh(
    core_axis_name="c", subcore_axis_name="s", num_cores=NC, num_subcores=NS
)


def _compute_chunk(h_hbm, acc, e_loc, ed_v, wn_v, rows_v):
    for v in range(K // LANES):
        sl = pl.ds(v * LANES, LANES)
        es = plsc.load_gather(e_loc, [ed_v[0, sl]])
        ed = plsc.load_gather(e_loc, [ed_v[1, sl]])
        ew = plsc.bitcast(ed_v[2, sl], jnp.float32)
        wn_v[sl] = ew / (1.0 + jnp.exp(es + ed))

    @plsc.parallel_loop(0, K, unroll=2)
    def _scale(k):
        wk = plsc.load_gather(wn_v, [jnp.full((LANES,), k, jnp.int32)])
        for j in range(D // LANES):
            sl2 = pl.ds(j * LANES, LANES)
            rows_v[k, sl2] = rows_v[k, sl2] * wk

    pltpu.sync_copy(rows_v, acc.at[ed_v.at[1]], add=True)


def _sc_body(h_hbm, e_hbm, edata_hbm, zeros_hbm, out_hbm,
             e_loc, ed0, ed1, wn_v, rows0, rows1, acc, sem0, sem1):
    cid = lax.axis_index("c")
    sid = lax.axis_index("s")
    wid = cid * NS + sid

    pltpu.sync_copy(zeros_hbm, acc.at[pl.ds(sid * ROWS_PER_SUB, ROWS_PER_SUB)])

    pltpu.sync_copy(e_hbm, e_loc)

    @pl.loop(0, EPAD // LANES)
    def _clip(i):
        sl = pl.ds(i * LANES, LANES)
        e_loc[sl] = jnp.minimum(jnp.maximum(e_loc[sl], 0.0), 10.0)

    plsc.subcore_barrier()

    cbase = wid * NCHUNK

    pltpu.sync_copy(edata_hbm.at[cbase], ed0)
    g0 = pltpu.async_copy(h_hbm.at[ed0.at[0]], rows0, sem0)

    @pl.loop(0, NCHUNK - 1, step=2)
    def _pair(c):
        pltpu.sync_copy(edata_hbm.at[cbase + c + 1], ed1)
        pltpu.make_async_copy(h_hbm.at[ed0.at[0]], rows0, sem0).wait()
        pltpu.async_copy(h_hbm.at[ed1.at[0]], rows1, sem1)
        _compute_chunk(h_hbm, acc, e_loc, ed0, wn_v, rows0)

        pltpu.sync_copy(edata_hbm.at[cbase + c + 2], ed0)
        pltpu.make_async_copy(h_hbm.at[ed1.at[0]], rows1, sem1).wait()
        pltpu.async_copy(h_hbm.at[ed0.at[0]], rows0, sem0)
        _compute_chunk(h_hbm, acc, e_loc, ed1, wn_v, rows1)

    pltpu.make_async_copy(h_hbm.at[ed0.at[0]], rows0, sem0).wait()
    _compute_chunk(h_hbm, acc, e_loc, ed0, wn_v, rows0)

    plsc.subcore_barrier()

    sl = pl.ds(sid * ROWS_PER_SUB, ROWS_PER_SUB)
    pltpu.sync_copy(acc.at[sl], out_hbm.at[cid].at[sl])


_SC_COMPILER_PARAMS = pltpu.CompilerParams()
if "needs_layout_passes" in pltpu.CompilerParams.__dataclass_fields__:
    _SC_COMPILER_PARAMS = dataclasses.replace(
        _SC_COMPILER_PARAMS, needs_layout_passes=False
    )

_sc_call = pl.kernel(
    _sc_body,
    out_type=jax.ShapeDtypeStruct((NC, NPAD, D), jnp.float32),
    mesh=_mesh,
    scratch_types=[
        pltpu.VMEM((EPAD,), jnp.float32),
        pltpu.VMEM((3, K), jnp.int32),
        pltpu.VMEM((3, K), jnp.int32),
        pltpu.VMEM((K,), jnp.float32),
        pltpu.VMEM((K, D), jnp.float32),
        pltpu.VMEM((K, D), jnp.float32),
        pltpu.VMEM_SHARED((NPAD, D), jnp.float32),
        pltpu.SemaphoreType.DMA,
        pltpu.SemaphoreType.DMA,
    ],
    compiler_params=_SC_COMPILER_PARAMS,
)


_MM_BLK = 1000


def _matmul_body(x_ref, wt_ref, b_ref, h_ref):
    h_ref[...] = (
        jnp.dot(x_ref[...], wt_ref[...], preferred_element_type=jnp.float32)
        + b_ref[...]
    )


_matmul = pl.pallas_call(
    _matmul_body,
    grid=(N // _MM_BLK,),
    in_specs=[
        pl.BlockSpec((_MM_BLK, D), lambda i: (i, 0)),
        pl.BlockSpec((D, D), lambda i: (0, 0)),
        pl.BlockSpec((1, D), lambda i: (0, 0)),
    ],
    out_specs=pl.BlockSpec((_MM_BLK, D), lambda i: (i, 0)),
    out_shape=jax.ShapeDtypeStruct((N, D), jnp.float32),
)


def _combine_body(p_ref, o_ref):
    o_ref[...] = p_ref[0] + p_ref[1]


_CB_BLK = 1024

_combine = pl.pallas_call(
    _combine_body,
    grid=(NPAD // _CB_BLK,),
    in_specs=[pl.BlockSpec((NC, _CB_BLK, D), lambda i: (0, i, 0))],
    out_specs=pl.BlockSpec((_CB_BLK, D), lambda i: (i, 0)),
    out_shape=jax.ShapeDtypeStruct((NPAD, D), jnp.float32),
)


@jax.jit
def kernel(x, edge_index, edge_weight, energy, W, b):
    src = edge_index[0].astype(jnp.int32)
    dst = edge_index[1].astype(jnp.int32)
    ew_bits = lax.bitcast_convert_type(edge_weight.astype(jnp.float32),
                                       jnp.int32)
    edata = jnp.stack(
        [src.reshape(ECHUNK, K), dst.reshape(ECHUNK, K),
         ew_bits.reshape(ECHUNK, K)], axis=1)
    e_pad = jnp.pad(energy.reshape(-1), (0, EPAD - N))
    zeros = jnp.zeros((ROWS_PER_SUB, D), jnp.float32)
    h = _matmul(x, W.T, b.reshape(1, D))
    partials = _sc_call(h, e_pad, edata, zeros)
    return _combine(partials)[:N]


# speedup vs baseline: 27.6090x; 1.1836x over previous
import dataclasses

import jax
import jax.numpy as jnp
from jax import lax
from jax.experimental import pallas as pl
from jax.experimental.pallas import tpu as pltpu
from jax.experimental.pallas import tpu_sc as plsc

N = 10000
E = 320000
D = 128

NC = 2
NS = 16
LANES = 16
NW = NC * NS
EPW = E // NW
K = 80
NCHUNK = EPW // K
ECHUNK = E // K
NPAD = 10240
ROWS_PER_SUB = NPAD // NS
EPAD = NPAD

_mesh = plsc.VectorSubcoreMesh(
    core_axis_name="c", subcore_axis_name="s", num_cores=NC, num_subcores=NS
)


def _sc_body(h_hbm, e_hbm, edata_hbm, zeros_hbm, out_hbm,
             e_loc, ed0, ed1, ed2, wn_v, rows0, rows1, rows2, acc,
             g0, g1, g2, s0, s1, s2m):
    ed = (ed0, ed1, ed2)
    rows = (rows0, rows1, rows2)
    gsems = (g0, g1, g2)
    ssems = (s0, s1, s2m)
    cid = lax.axis_index("c")
    sid = lax.axis_index("s")
    wid = cid * NS + sid

    pltpu.sync_copy(zeros_hbm, acc.at[pl.ds(sid * ROWS_PER_SUB, ROWS_PER_SUB)])

    pltpu.sync_copy(e_hbm, e_loc)

    @pl.loop(0, EPAD // LANES)
    def _clip(i):
        sl = pl.ds(i * LANES, LANES)
        e_loc[sl] = jnp.minimum(jnp.maximum(e_loc[sl], 0.0), 10.0)

    plsc.subcore_barrier()

    cbase = wid * NCHUNK

    def half(c, b, prefetch, first):
        b2 = (b + 2) % 3
        pltpu.make_async_copy(h_hbm.at[ed[b].at[0]], rows[b],
                              gsems[b]).wait()

        for v in range(K // LANES):
            sl = pl.ds(v * LANES, LANES)
            es = plsc.load_gather(e_loc, [ed[b][0, sl]])
            edv = plsc.load_gather(e_loc, [ed[b][1, sl]])
            ew = plsc.bitcast(ed[b][2, sl], jnp.float32)
            wn_v[sl] = ew / (1.0 + jnp.exp(es + edv))

        rv = rows[b]

        @plsc.parallel_loop(0, K, unroll=4)
        def _scale(k):
            wk = plsc.load_gather(wn_v, [jnp.full((LANES,), k, jnp.int32)])
            for j in range(D // LANES):
                sl2 = pl.ds(j * LANES, LANES)
                rv[k, sl2] = rv[k, sl2] * wk

        pltpu.async_copy(rows[b], acc.at[ed[b].at[1]], ssems[b], add=True)

        if prefetch:
            if not first:
                pltpu.make_async_copy(rows[b2], acc.at[ed[b2].at[1]],
                                      ssems[b2]).wait()
            pltpu.sync_copy(edata_hbm.at[cbase + c + 2], ed[b2])
            pltpu.async_copy(h_hbm.at[ed[b2].at[0]], rows[b2], gsems[b2])

    pltpu.sync_copy(edata_hbm.at[cbase], ed[0])
    pltpu.sync_copy(edata_hbm.at[cbase + 1], ed[1])
    pltpu.async_copy(h_hbm.at[ed[0].at[0]], rows[0], gsems[0])
    pltpu.async_copy(h_hbm.at[ed[1].at[0]], rows[1], gsems[1])
    half(0, 0, prefetch=True, first=True)
    half(1, 1, prefetch=True, first=False)

    @pl.loop(2, NCHUNK - 3, step=3)
    def _trip(c):
        for q in range(3):
            half(c + q, (2 + q) % 3, prefetch=True, first=False)

    half(NCHUNK - 3, (NCHUNK - 3) % 3, prefetch=True, first=False)
    half(NCHUNK - 2, (NCHUNK - 2) % 3, prefetch=False, first=False)
    half(NCHUNK - 1, (NCHUNK - 1) % 3, prefetch=False, first=False)

    for c in (NCHUNK - 3, NCHUNK - 2, NCHUNK - 1):
        b = c % 3
        pltpu.make_async_copy(rows[b], acc.at[ed[b].at[1]],
                              ssems[b]).wait()

    plsc.subcore_barrier()

    sl = pl.ds(sid * ROWS_PER_SUB, ROWS_PER_SUB)
    pltpu.sync_copy(acc.at[sl], out_hbm.at[cid].at[sl])


_SC_COMPILER_PARAMS = pltpu.CompilerParams()
if "needs_layout_passes" in pltpu.CompilerParams.__dataclass_fields__:
    _SC_COMPILER_PARAMS = dataclasses.replace(
        _SC_COMPILER_PARAMS, needs_layout_passes=False
    )

_sc_call = pl.kernel(
    _sc_body,
    out_type=jax.ShapeDtypeStruct((NC, NPAD, D), jnp.float32),
    mesh=_mesh,
    scratch_types=[
        pltpu.VMEM((EPAD,), jnp.float32),
        pltpu.VMEM((3, K), jnp.int32),
        pltpu.VMEM((3, K), jnp.int32),
        pltpu.VMEM((3, K), jnp.int32),
        pltpu.VMEM((K,), jnp.float32),
        pltpu.VMEM((K, D), jnp.float32),
        pltpu.VMEM((K, D), jnp.float32),
        pltpu.VMEM((K, D), jnp.float32),
        pltpu.VMEM_SHARED((NPAD, D), jnp.float32),
        pltpu.SemaphoreType.DMA,
        pltpu.SemaphoreType.DMA,
        pltpu.SemaphoreType.DMA,
        pltpu.SemaphoreType.DMA,
        pltpu.SemaphoreType.DMA,
        pltpu.SemaphoreType.DMA,
    ],
    compiler_params=_SC_COMPILER_PARAMS,
)


_MM_BLK = 1000


def _matmul_body(x_ref, wt_ref, b_ref, h_ref):
    h_ref[...] = (
        jnp.dot(x_ref[...], wt_ref[...], preferred_element_type=jnp.float32)
        + b_ref[...]
    )


_matmul = pl.pallas_call(
    _matmul_body,
    grid=(N // _MM_BLK,),
    in_specs=[
        pl.BlockSpec((_MM_BLK, D), lambda i: (i, 0)),
        pl.BlockSpec((D, D), lambda i: (0, 0)),
        pl.BlockSpec((1, D), lambda i: (0, 0)),
    ],
    out_specs=pl.BlockSpec((_MM_BLK, D), lambda i: (i, 0)),
    out_shape=jax.ShapeDtypeStruct((N, D), jnp.float32),
)


def _combine_body(p_ref, o_ref):
    o_ref[...] = p_ref[0] + p_ref[1]


_CB_BLK = 1024

_combine = pl.pallas_call(
    _combine_body,
    grid=(NPAD // _CB_BLK,),
    in_specs=[pl.BlockSpec((NC, _CB_BLK, D), lambda i: (0, i, 0))],
    out_specs=pl.BlockSpec((_CB_BLK, D), lambda i: (i, 0)),
    out_shape=jax.ShapeDtypeStruct((NPAD, D), jnp.float32),
)


@jax.jit
def kernel(x, edge_index, edge_weight, energy, W, b):
    src = edge_index[0].astype(jnp.int32)
    dst = edge_index[1].astype(jnp.int32)
    ew_bits = lax.bitcast_convert_type(edge_weight.astype(jnp.float32),
                                       jnp.int32)
    edata = jnp.stack(
        [src.reshape(ECHUNK, K), dst.reshape(ECHUNK, K),
         ew_bits.reshape(ECHUNK, K)], axis=1)
    e_pad = jnp.pad(energy.reshape(-1), (0, EPAD - N))
    zeros = jnp.zeros((ROWS_PER_SUB, D), jnp.float32)
    h = _matmul(x, W.T, b.reshape(1, D))
    partials = _sc_call(h, e_pad, edata, zeros)
    return _combine(partials)[:N]
